# baseline (speedup 1.0000x reference)
import jax
import jax.numpy as jnp
from jax import lax
from jax.experimental import pallas as pl
from jax.experimental.pallas import tpu as pltpu
from jax.experimental.pallas import tpu_sc as plsc


M, D, B = 100000, 128, 16384
NC, NS, L = 2, 16, 16
H = M // NC
NR = 5
RS = H // NR
ROWS_T = (RS // NS) // 8 * 8
ROWS_REM = RS - NS * ROWS_T
BT = B // NS
DUMP = RS
TBL = RS + 8
CTBL = TBL // 8 + 2
CH = 64
NG = CH // L
TRASH = BT + CH
CBN = BT + CH + 8


def _msg_body(val_ref, rt_ref, w_ref, b_ref, o_ref):
    rt = rt_ref[...][:, None]
    w = w_ref[...][None, :]
    b = b_ref[...][None, :]
    o_ref[...] = val_ref[...] + jnp.cos(rt * w + b)


def _compute_msg(val, rt, w, b):
    blk = 2048
    grid = (B // blk,)
    return pl.pallas_call(
        _msg_body,
        grid=grid,
        in_specs=[
            pl.BlockSpec((blk, D), lambda i: (i, 0)),
            pl.BlockSpec((blk,), lambda i: (i,)),
            pl.BlockSpec((D,), lambda i: (0,)),
            pl.BlockSpec((D,), lambda i: (0,)),
        ],
        out_specs=pl.BlockSpec((blk, D), lambda i: (i, 0)),
        out_shape=jax.ShapeDtypeStruct((B, D), jnp.float32),
    )(val, rt, w, b)


def _sc_body(mem_h, idx_h, msg_h, out_h,
             sums_sh, cnt_sh, idx_t, cb_mid, cb_lid,
             zidx, cidx, gidx, midx, mrow, crow, ebuf, scanb, csem):
    cid = lax.axis_index("c")
    sid = lax.axis_index("s")
    lane = lax.iota(jnp.int32, L)
    zf = jnp.zeros((L,), jnp.float32)
    zi = jnp.zeros((L,), jnp.int32)
    onesf = jnp.ones((L,), jnp.float32)
    for r in range(CH):
        for v in range(D // L):
            ebuf[r, pl.ds(v * L, L)] = zf

    pltpu.sync_copy(idx_h.at[pl.ds(sid * BT, BT)], idx_t)

    copies = []
    for rr in range(NR):
        base = cid * H + rr * RS
        rlo = base + sid * ROWS_T
        copies.append(pltpu.async_copy(
            mem_h.at[pl.ds(rlo, ROWS_T)], out_h.at[pl.ds(rlo, ROWS_T)],
            csem.at[rr]))

        @pl.when(sid == 0)
        def _copy_rem():
            lo = base + NS * ROWS_T
            pltpu.sync_copy(mem_h.at[pl.ds(lo, ROWS_REM)],
                            out_h.at[pl.ds(lo, ROWS_REM)])

    for rr in range(NR):
        base = cid * H + rr * RS

        def pf(j, c):
            cb_lid[pl.ds(j * L, L)] = jnp.full((L,), DUMP, jnp.int32)
            cb_mid[pl.ds(j * L, L)] = zi
            return c
        lax.fori_loop(0, (BT + CH) // L, pf, 0)

        def cp(j, off):
            v = idx_t[pl.ds(j * L, L)]
            m = (v >= base) & (v < base + RS)
            mi = jnp.where(m, jnp.int32(1), jnp.int32(0))
            s = mi
            for kk in (1, 2, 4, 8):
                scanb[...] = s
                g = plsc.load_gather(scanb, [jnp.maximum(lane - kk, 0)])
                s = s + jnp.where(lane >= kk, g, jnp.int32(0))
            pos = jnp.where(m, off + s - 1, jnp.int32(TRASH))
            plsc.store_scatter(cb_mid, [pos], sid * BT + j * L + lane)
            plsc.store_scatter(cb_lid, [pos], v - base)
            pc = plsc.all_reduce_population_count(m)
            return off + pc[0]
        k = lax.fori_loop(0, BT // L, cp, jnp.int32(0))
        nch = lax.shift_right_logical(k + CH - 1, 6)

        def pz(c, u):
            for g in range(NG):
                vlid = cb_lid[pl.ds(c * CH + g * L, L)]
                zidx[pl.ds(g * L, L)] = vlid
                cidx[pl.ds(g * L, L)] = lax.shift_right_logical(vlid, 3)
            pltpu.sync_copy(ebuf, sums_sh.at[zidx])
            pltpu.sync_copy(ebuf, cnt_sh.at[cidx])
            return u
        lax.fori_loop(0, nch, pz, 0)
        plsc.subcore_barrier()

        def pc(c, u):
            for g in range(NG):
                vlid = cb_lid[pl.ds(c * CH + g * L, L)]
                cidx[pl.ds(g * L, L)] = lax.shift_right_logical(vlid, 3)
                col = (vlid & jnp.int32(7)) * L
                plsc.store_scatter(ebuf, [g * L + lane, col], onesf)
            pltpu.sync_copy(ebuf, cnt_sh.at[cidx], add=True)
            for g in range(NG):
                vlid = cb_lid[pl.ds(c * CH + g * L, L)]
                col = (vlid & jnp.int32(7)) * L
                plsc.store_scatter(ebuf, [g * L + lane, col], zf)
            return u
        lax.fori_loop(0, nch, pc, 0)
        plsc.subcore_barrier()

        def ps(c, u):
            for g in range(NG):
                vlid = cb_lid[pl.ds(c * CH + g * L, L)]
                zidx[pl.ds(g * L, L)] = vlid
                cidx[pl.ds(g * L, L)] = lax.shift_right_logical(vlid, 3)
                midx[pl.ds(g * L, L)] = cb_mid[pl.ds(c * CH + g * L, L)]
            pltpu.sync_copy(msg_h.at[midx], mrow)
            pltpu.sync_copy(cnt_sh.at[cidx], crow)
            for g in range(NG):
                vlid = cb_lid[pl.ds(c * CH + g * L, L)]
                cnt = plsc.load_gather(
                    crow, [g * L + lane, (vlid & jnp.int32(7)) * L])
                rcp = 1.0 / jnp.maximum(cnt, 1.0)
                for r in range(L):
                    br = rcp[r]
                    for v in range(D // L):
                        sl = pl.ds(v * L, L)
                        mrow[g * L + r, sl] = mrow[g * L + r, sl] * br
            pltpu.sync_copy(mrow, sums_sh.at[zidx], add=True)
            return u
        lax.fori_loop(0, nch, ps, 0)

        copies[rr].wait()
        plsc.subcore_barrier()

        l0 = cb_lid[pl.ds(0, L)][0]

        def pd(c, u):
            for g in range(NG):
                v = cb_lid[pl.ds(c * CH + g * L, L)]
                vd = jnp.where((c * CH + g * L + lane) < k, v, zi + l0)
                zidx[pl.ds(g * L, L)] = vd
                gidx[pl.ds(g * L, L)] = vd + base
            pltpu.sync_copy(sums_sh.at[zidx], mrow)
            pltpu.sync_copy(mrow, out_h.at[gidx])
            return u
        lax.fori_loop(0, nch, pd, 0)
        plsc.subcore_barrier()


def _SCRATCH():
    return [
        pltpu.VMEM_SHARED((TBL, D), jnp.float32),
        pltpu.VMEM_SHARED((CTBL, D), jnp.float32),
        pltpu.VMEM((BT,), jnp.int32),
        pltpu.VMEM((CBN,), jnp.int32),
        pltpu.VMEM((CBN,), jnp.int32),
        pltpu.VMEM((CH,), jnp.int32),
        pltpu.VMEM((CH,), jnp.int32),
        pltpu.VMEM((CH,), jnp.int32),
        pltpu.VMEM((CH,), jnp.int32),
        pltpu.VMEM((CH, D), jnp.float32),
        pltpu.VMEM((CH, D), jnp.float32),
        pltpu.VMEM((CH, D), jnp.float32),
        pltpu.VMEM((L,), jnp.int32),
        pltpu.SemaphoreType.DMA((NR,)),
    ]


def _scatter_sc(mem, idx, msg):
    mesh = plsc.VectorSubcoreMesh(core_axis_name="c", subcore_axis_name="s")
    f = pl.kernel(
        _sc_body,
        out_type=jax.ShapeDtypeStruct((M, D), jnp.float32),
        mesh=mesh,
        compiler_params=pltpu.CompilerParams(needs_layout_passes=False),
        scratch_types=_SCRATCH(),
    )
    return f(mem, idx, msg)


def kernel(mem, idx, val, edge_times, relative_times, w, b):
    msg = _compute_msg(val, relative_times, w, b)
    return _scatter_sc(mem, idx, msg)


# speedup vs baseline: 6.2605x; 6.2605x over previous
import jax
import jax.numpy as jnp
from jax import lax
from jax.experimental import pallas as pl
from jax.experimental.pallas import tpu as pltpu
from jax.experimental.pallas import tpu_sc as plsc


M, D, B = 100000, 128, 16384
NC, NS, L = 2, 16, 16
H = M // NC
NR = 5
RS = H // NR
ROWS_T = (RS // NS) // 8 * 8
ROWS_REM = RS - NS * ROWS_T
BT = B // NS
DUMP = RS
TBL = RS + 8
CTBL = TBL // 8 + 2
CH = 64
NG = CH // L
TRASH = BT + CH
CBN = BT + CH + 8


def _msg_body(val_ref, rt_ref, w_ref, b_ref, o_ref):
    rt = rt_ref[...][:, None]
    w = w_ref[...][None, :]
    b = b_ref[...][None, :]
    o_ref[...] = val_ref[...] + jnp.cos(rt * w + b)


def _compute_msg(val, rt, w, b):
    blk = 2048
    grid = (B // blk,)
    return pl.pallas_call(
        _msg_body,
        grid=grid,
        in_specs=[
            pl.BlockSpec((blk, D), lambda i: (i, 0)),
            pl.BlockSpec((blk,), lambda i: (i,)),
            pl.BlockSpec((D,), lambda i: (0,)),
            pl.BlockSpec((D,), lambda i: (0,)),
        ],
        out_specs=pl.BlockSpec((blk, D), lambda i: (i, 0)),
        out_shape=jax.ShapeDtypeStruct((B, D), jnp.float32),
    )(val, rt, w, b)


def _sc_body(out_h, idx_h, msg_h,
             sums_sh, cnt_sh, idx_t, cb_mid, cb_lid,
             zidx, cidx, gidx, midx, mrow, crow, ebuf, scanb, dsem):
    cid = lax.axis_index("c")
    sid = lax.axis_index("s")
    lane = lax.iota(jnp.int32, L)
    zf = jnp.zeros((L,), jnp.float32)
    zi = jnp.zeros((L,), jnp.int32)
    onesf = jnp.ones((L,), jnp.float32)
    for r in range(CH):
        for v in range(D // L):
            ebuf[r, pl.ds(v * L, L)] = zf

    pltpu.sync_copy(idx_h.at[pl.ds(sid * BT, BT)], idx_t)

    for rr in range(NR):
        base = cid * H + rr * RS

        def pf(j, c):
            cb_lid[pl.ds(j * L, L)] = jnp.full((L,), DUMP, jnp.int32)
            cb_mid[pl.ds(j * L, L)] = zi
            return c
        lax.fori_loop(0, (BT + CH) // L, pf, 0)

        def cp(j, off):
            v = idx_t[pl.ds(j * L, L)]
            m = (v >= base) & (v < base + RS)
            mi = jnp.where(m, jnp.int32(1), jnp.int32(0))
            s = mi
            for kk in (1, 2, 4, 8):
                scanb[...] = s
                g = plsc.load_gather(scanb, [jnp.maximum(lane - kk, 0)])
                s = s + jnp.where(lane >= kk, g, jnp.int32(0))
            pos = jnp.where(m, off + s - 1, jnp.int32(TRASH))
            plsc.store_scatter(cb_mid, [pos], sid * BT + j * L + lane)
            plsc.store_scatter(cb_lid, [pos], v - base)
            pc = plsc.all_reduce_population_count(m)
            return off + pc[0]
        k = lax.fori_loop(0, BT // L, cp, jnp.int32(0))
        nch = lax.shift_right_logical(k + CH - 1, 6)

        def pz(c, u):
            for g in range(NG):
                vlid = cb_lid[pl.ds(c * CH + g * L, L)]
                zidx[pl.ds(g * L, L)] = vlid
                cidx[pl.ds(g * L, L)] = lax.shift_right_logical(vlid, 3)
            a = pltpu.async_copy(ebuf, sums_sh.at[zidx], dsem.at[0])
            b = pltpu.async_copy(ebuf, cnt_sh.at[cidx], dsem.at[1])
            a.wait()
            b.wait()
            return u
        lax.fori_loop(0, nch, pz, 0)
        plsc.subcore_barrier()

        def pc(c, u):
            for g in range(NG):
                vlid = cb_lid[pl.ds(c * CH + g * L, L)]
                cidx[pl.ds(g * L, L)] = lax.shift_right_logical(vlid, 3)
                col = (vlid & jnp.int32(7)) * L
                plsc.store_scatter(ebuf, [g * L + lane, col], onesf)
            pltpu.sync_copy(ebuf, cnt_sh.at[cidx], add=True)
            for g in range(NG):
                vlid = cb_lid[pl.ds(c * CH + g * L, L)]
                col = (vlid & jnp.int32(7)) * L
                plsc.store_scatter(ebuf, [g * L + lane, col], zf)
            return u
        lax.fori_loop(0, nch, pc, 0)
        plsc.subcore_barrier()

        def ps(c, u):
            for g in range(NG):
                vlid = cb_lid[pl.ds(c * CH + g * L, L)]
                zidx[pl.ds(g * L, L)] = vlid
                cidx[pl.ds(g * L, L)] = lax.shift_right_logical(vlid, 3)
                midx[pl.ds(g * L, L)] = cb_mid[pl.ds(c * CH + g * L, L)]
            a = pltpu.async_copy(msg_h.at[midx], mrow, dsem.at[0])
            b = pltpu.async_copy(cnt_sh.at[cidx], crow, dsem.at[1])
            a.wait()
            b.wait()
            for g in range(NG):
                vlid = cb_lid[pl.ds(c * CH + g * L, L)]
                cnt = plsc.load_gather(
                    crow, [g * L + lane, (vlid & jnp.int32(7)) * L])
                rcp = 1.0 / jnp.maximum(cnt, 1.0)
                for r in range(L):
                    br = rcp[r]

                    @pl.when(br < 1.0)
                    def _scale(g=g, r=r, br=br):
                        for v in range(D // L):
                            sl = pl.ds(v * L, L)
                            mrow[g * L + r, sl] = mrow[g * L + r, sl] * br
            pltpu.sync_copy(mrow, sums_sh.at[zidx], add=True)
            return u
        lax.fori_loop(0, nch, ps, 0)
        plsc.subcore_barrier()

        l0 = cb_lid[pl.ds(0, L)][0]

        def pd(c, u):
            for g in range(NG):
                v = cb_lid[pl.ds(c * CH + g * L, L)]
                vd = jnp.where((c * CH + g * L + lane) < k, v, zi + l0)
                zidx[pl.ds(g * L, L)] = vd
                gidx[pl.ds(g * L, L)] = vd + base
            pltpu.sync_copy(sums_sh.at[zidx], mrow)
            pltpu.sync_copy(mrow, out_h.at[gidx])
            return u
        lax.fori_loop(0, nch, pd, 0)
        plsc.subcore_barrier()


def _SCRATCH():
    return [
        pltpu.VMEM_SHARED((TBL, D), jnp.float32),
        pltpu.VMEM_SHARED((CTBL, D), jnp.float32),
        pltpu.VMEM((BT,), jnp.int32),
        pltpu.VMEM((CBN,), jnp.int32),
        pltpu.VMEM((CBN,), jnp.int32),
        pltpu.VMEM((CH,), jnp.int32),
        pltpu.VMEM((CH,), jnp.int32),
        pltpu.VMEM((CH,), jnp.int32),
        pltpu.VMEM((CH,), jnp.int32),
        pltpu.VMEM((CH, D), jnp.float32),
        pltpu.VMEM((CH, D), jnp.float32),
        pltpu.VMEM((CH, D), jnp.float32),
        pltpu.VMEM((L,), jnp.int32),
        pltpu.SemaphoreType.DMA((2,)),
    ]


def _scatter_sc(out_ref, idx, msg):
    mesh = plsc.VectorSubcoreMesh(core_axis_name="c", subcore_axis_name="s")
    f = pl.kernel(
        _sc_body,
        out_type=(),
        mesh=mesh,
        compiler_params=pltpu.CompilerParams(needs_layout_passes=False),
        scratch_types=_SCRATCH(),
    )
    f(out_ref, idx, msg)


def kernel(mem, idx, val, edge_times, relative_times, w, b):
    msg = _compute_msg(val, relative_times, w, b)
    out_ref = jax.new_ref(jnp.copy(mem))
    _scatter_sc(out_ref, idx, msg)
    return out_ref[...]
